# Initial kernel scaffold; baseline (speedup 1.0000x reference)
import jax
import jax.numpy as jnp
from jax.experimental import pallas as pl


def kernel(x, edge_index, query, W_l0, b_l0, W_r0, W_l1, b_l1, W_r1, W_l2, b_l2, W_r2, W_lin, b_lin):
    raise NotImplementedError("write your pallas kernel here")


# speedup vs baseline: 1.6332x; 1.6332x over previous
import functools

import jax
import jax.numpy as jnp
from jax import lax
from jax.experimental import pallas as pl
from jax.experimental.pallas import tpu as pltpu
from jax.experimental.pallas import tpu_sc as plsc

N = 10000
D = 128
E = 320000

NC = 2
NS = 16
NW = NC * NS
NP = 10240
RPW = NP // NW
TRASH = RPW
ACC_ROWS = RPW + 16

C = 16000
G = 32
ROWCAP = E + C + G + (E // C) * 16 + 32

_mesh = plsc.VectorSubcoreMesh(core_axis_name="c", subcore_axis_name="s")


def _wid():
    return lax.axis_index("s") * NC + lax.axis_index("c")


@functools.partial(
    pl.kernel,
    out_type=(
        jax.ShapeDtypeStruct((NW * ROWCAP,), jnp.int32),
        jax.ShapeDtypeStruct((NW * 16,), jnp.int32),
    ),
    mesh=_mesh,
    scratch_types=[
        pltpu.VMEM((C,), jnp.int32),
        pltpu.VMEM((C,), jnp.int32),
        pltpu.VMEM((C + 16,), jnp.int32),
        pltpu.VMEM((16,), jnp.int32),
    ],
    compiler_params=pltpu.CompilerParams(needs_layout_passes=False),
)
def _partition(src_hbm, dst_hbm, packed_hbm, counts_hbm, srcv, dstv, mbuf, cntv):
    w = _wid()
    lo = w * RPW
    hi = lo + RPW
    wbase = pl.multiple_of(w * ROWCAP, 32)
    trash_vec = jnp.full((16,), TRASH, jnp.int32)

    def filt(i, cnt):
        s = srcv[pl.ds(i * 16, 16)]
        d = dstv[pl.ds(i * 16, 16)]
        m = jnp.logical_and(d >= lo, d < hi)
        packed = lax.shift_left(s, 9) + (d - lo)
        key = jnp.where(m, 0, 1).astype(jnp.int32)
        _, sv = plsc.sort_key_val(key, packed)
        mbuf[pl.ds(cnt, 16)] = sv
        pc = plsc.all_reduce_population_count(m)
        return cnt + pc[0]

    def chunk(ci, total):
        pltpu.sync_copy(src_hbm.at[pl.ds(ci * C, C)], srcv)
        pltpu.sync_copy(dst_hbm.at[pl.ds(ci * C, C)], dstv)
        cnt = lax.fori_loop(0, C // 16, filt, 0)
        mbuf[pl.ds(cnt, 16)] = trash_vec
        cnt16 = ((cnt + 15) // 16) * 16
        off = pl.multiple_of(wbase + total, 16)
        pltpu.sync_copy(mbuf.at[pl.ds(0, C)], packed_hbm.at[pl.ds(off, C)])
        return total + cnt16

    total = lax.fori_loop(0, E // C, chunk, 0)
    for j in range(G // 16):
        mbuf[pl.ds(16 * j, 16)] = trash_vec
    off = pl.multiple_of(wbase + total, 16)
    pltpu.sync_copy(mbuf.at[pl.ds(0, G)], packed_hbm.at[pl.ds(off, G)])
    nb = (total + G - 1) // G
    cntv[pl.ds(0, 16)] = jnp.full((16,), 0, jnp.int32) + nb
    pltpu.sync_copy(cntv, counts_hbm.at[pl.ds(pl.multiple_of(w * 16, 16), 16)])


@functools.partial(
    pl.kernel,
    out_type=jax.ShapeDtypeStruct((NP, D), jnp.float32),
    mesh=_mesh,
    scratch_types=[
        pltpu.VMEM((ACC_ROWS, D), jnp.float32),
        pltpu.VMEM((G + 16,), jnp.int32),
        pltpu.VMEM((G,), jnp.int32),
        pltpu.VMEM((G, D), jnp.float32),
        pltpu.VMEM((16,), jnp.int32),
        pltpu.SemaphoreType.DMA,
    ],
    compiler_params=pltpu.CompilerParams(needs_layout_passes=False),
)
def _maxagg(h_hbm, packed_hbm, counts_hbm, out_hbm, acc, pk, idx, rows, cntv, sem):
    w = _wid()
    lo = pl.multiple_of(w * RPW, 32)
    wbase = pl.multiple_of(w * ROWCAP, 32)
    neg = jnp.full((16,), -3e38, jnp.float32)

    def initrow(r, _):
        for k in range(D // 16):
            acc[r, pl.ds(16 * k, 16)] = neg
        return 0

    lax.fori_loop(0, ACC_ROWS, initrow, 0)

    pltpu.sync_copy(counts_hbm.at[pl.ds(pl.multiple_of(w * 16, 16), 16)], cntv)
    nb = cntv[pl.ds(0, 16)][0]

    def batch(g, _):
        off = pl.multiple_of(wbase + g * G, 32)
        pltpu.sync_copy(packed_hbm.at[pl.ds(off, G)], pk.at[pl.ds(0, G)])
        for j in range(G // 16):
            v = pk[pl.ds(16 * j, 16)]
            idx[pl.ds(16 * j, 16)] = lax.shift_right_logical(v, 9)
        pltpu.async_copy(h_hbm.at[idx], rows, sem).wait()

        def drain(r, _):
            d = lax.bitwise_and(pk[pl.ds(r, 16)], 511)[0]
            for k in range(D // 16):
                sl = pl.ds(16 * k, 16)
                acc[d, sl] = jnp.maximum(acc[d, sl], rows[r, sl])
            return 0

        lax.fori_loop(0, G, drain, 0)
        return 0

    lax.fori_loop(0, nb, batch, 0)

    def fin(r, _):
        for k in range(D // 16):
            sl = pl.ds(16 * k, 16)
            v = acc[r, sl]
            acc[r, sl] = jnp.where(v > -1e37, v, 0.0)
        return 0

    lax.fori_loop(0, RPW, fin, 0)
    pltpu.sync_copy(acc.at[pl.ds(0, RPW)], out_hbm.at[pl.ds(lo, RPW)])


_R = 1024


def _dense_body(agg_ref, h_ref, wlT_ref, bl_ref, wrT_ref, o_ref):
    a = jnp.dot(agg_ref[...], wlT_ref[...], preferred_element_type=jnp.float32)
    b = jnp.dot(h_ref[...], wrT_ref[...], preferred_element_type=jnp.float32)
    o_ref[...] = jnp.maximum(a + b + bl_ref[...], 0.0)


_dense = pl.pallas_call(
    _dense_body,
    grid=(NP // _R,),
    in_specs=[
        pl.BlockSpec((_R, D), lambda i: (i, 0)),
        pl.BlockSpec((_R, D), lambda i: (i, 0)),
        pl.BlockSpec((D, D), lambda i: (0, 0)),
        pl.BlockSpec((1, D), lambda i: (0, 0)),
        pl.BlockSpec((D, D), lambda i: (0, 0)),
    ],
    out_specs=pl.BlockSpec((_R, D), lambda i: (i, 0)),
    out_shape=jax.ShapeDtypeStruct((NP, D), jnp.float32),
)


def _dec_body(h_ref, wT_ref, b_ref, q_ref, o_ref):
    t = jnp.dot(h_ref[...], wT_ref[...], preferred_element_type=jnp.float32)
    t = t + b_ref[...]
    q = q_ref[...]
    qn = q * lax.rsqrt(jnp.maximum(jnp.sum(q * q), 1e-24))
    num = jnp.sum(t * qn, axis=1, keepdims=True)
    den = jnp.sqrt(jnp.sum(t * t, axis=1, keepdims=True))
    o_ref[...] = num / jnp.maximum(den, 1e-12)


_decoder = pl.pallas_call(
    _dec_body,
    grid=(NP // _R,),
    in_specs=[
        pl.BlockSpec((_R, D), lambda i: (i, 0)),
        pl.BlockSpec((D, D), lambda i: (0, 0)),
        pl.BlockSpec((1, D), lambda i: (0, 0)),
        pl.BlockSpec((1, D), lambda i: (0, 0)),
    ],
    out_specs=pl.BlockSpec((_R, 1), lambda i: (i, 0)),
    out_shape=jax.ShapeDtypeStruct((NP, 1), jnp.float32),
)


def kernel(x, edge_index, query, W_l0, b_l0, W_r0, W_l1, b_l1, W_r1,
           W_l2, b_l2, W_r2, W_lin, b_lin):
    src = edge_index[0].astype(jnp.int32)
    dst = edge_index[1].astype(jnp.int32)
    h = jnp.concatenate([x, jnp.zeros((NP - N, D), jnp.float32)], axis=0)

    packed, counts = _partition(src, dst)

    for Wl, bl, Wr in ((W_l0, b_l0, W_r0), (W_l1, b_l1, W_r1), (W_l2, b_l2, W_r2)):
        agg = _maxagg(h, packed, counts)
        h = _dense(agg, h, Wl.T, bl.reshape(1, D), Wr.T)

    scores = _decoder(h, W_lin.T, b_lin.reshape(1, D), query.reshape(1, D))
    return scores[:N, 0]
